# speedup vs baseline: 3.7279x; 3.7279x over previous
import jax
import jax.numpy as jnp
from jax.experimental import pallas as pl


def _add_body(x_ref, t_ref, o_ref):
    o_ref[...] = x_ref[...] + t_ref[...]


def kernel(x, table):
    seq_len, batch, d_model = x.shape
    t3 = table[:seq_len].reshape(seq_len, 1, d_model)
    bs = 512
    grid = (seq_len // bs,)
    return pl.pallas_call(
        _add_body,
        grid=grid,
        in_specs=[
            pl.BlockSpec((bs, batch, d_model), lambda i: (i, 0, 0)),
            pl.BlockSpec((bs, 1, d_model), lambda i: (i, 0, 0)),
        ],
        out_specs=pl.BlockSpec((bs, batch, d_model), lambda i: (i, 0, 0)),
        out_shape=jax.ShapeDtypeStruct(x.shape, x.dtype),
    )(x, t3)
